# Initial kernel scaffold; baseline (speedup 1.0000x reference)
import jax
import jax.numpy as jnp
from jax.experimental import pallas as pl


def kernel(x, edge_index, batch, params):
    raise NotImplementedError("write your pallas kernel here")


# speedup vs baseline: 4.1773x; 4.1773x over previous
import functools

import jax
import jax.numpy as jnp
from jax import lax
from jax.experimental import pallas as pl
from jax.experimental.pallas import tpu as pltpu
from jax.experimental.pallas import tpu_sc as plsc

NTYPES = 100
AF = 128
HID = 256
HALF = 128
NLAYERS = 5
N_NODES = 10000
N_EDGES = 160000
NGRAPHS = 64

NTILES = 16
EPT = N_EDGES // NTILES
CHUNK = 80
NCHUNK = EPT // CHUNK
ROWS_A = 624
ROWS_LAST = N_NODES - (NTILES - 1) * ROWS_A

RB = 1000
NB = N_NODES // RB


def _sc_agg_body(row_hbm, col_hbm, ha_hbm, hb_hbm, outa_hbm, outb_hbm,
                 colv, rowv, gbuf, agg_sh, sem):
    c = lax.axis_index("c")
    s = lax.axis_index("s")

    def zrow(r, carry):
        for k in range(HALF // 16):
            gbuf[r, pl.ds(k * 16, 16)] = jnp.zeros((16,), jnp.float32)
        return carry
    lax.fori_loop(0, CHUNK, zrow, 0)
    base = s * ROWS_A
    for b in range(ROWS_A // CHUNK):
        pltpu.sync_copy(gbuf, agg_sh.at[pl.ds(base + b * CHUNK, CHUNK)])
    pltpu.sync_copy(gbuf.at[pl.ds(0, ROWS_A - 560)],
                    agg_sh.at[pl.ds(base + 560, ROWS_A - 560)])

    @pl.when(s == NTILES - 1)
    def _():
        pltpu.sync_copy(gbuf.at[pl.ds(0, ROWS_LAST - ROWS_A)],
                        agg_sh.at[pl.ds(base + ROWS_A, ROWS_LAST - ROWS_A)])
    plsc.subcore_barrier()

    pltpu.sync_copy(col_hbm.at[s], colv)
    pltpu.sync_copy(row_hbm.at[s], rowv)

    def chunk(j, carry):
        @pl.when(c == 0)
        def _():
            pltpu.async_copy(ha_hbm.at[colv.at[j]], gbuf, sem).wait()

        @pl.when(c == 1)
        def _():
            pltpu.async_copy(hb_hbm.at[colv.at[j]], gbuf, sem).wait()

        pltpu.sync_copy(gbuf, agg_sh.at[rowv.at[j]], add=True)
        return carry
    lax.fori_loop(0, NCHUNK, chunk, 0)
    plsc.subcore_barrier()

    @pl.when((c == 0) & (s != NTILES - 1))
    def _():
        pltpu.sync_copy(agg_sh.at[pl.ds(base, ROWS_A)],
                        outa_hbm.at[pl.ds(base, ROWS_A)])

    @pl.when((c == 0) & (s == NTILES - 1))
    def _():
        pltpu.sync_copy(agg_sh.at[pl.ds(base, ROWS_LAST)],
                        outa_hbm.at[pl.ds(base, ROWS_LAST)])

    @pl.when((c == 1) & (s != NTILES - 1))
    def _():
        pltpu.sync_copy(agg_sh.at[pl.ds(base, ROWS_A)],
                        outb_hbm.at[pl.ds(base, ROWS_A)])

    @pl.when((c == 1) & (s == NTILES - 1))
    def _():
        pltpu.sync_copy(agg_sh.at[pl.ds(base, ROWS_LAST)],
                        outb_hbm.at[pl.ds(base, ROWS_LAST)])


@functools.cache
def _sc_agg_kernel():
    return pl.kernel(
        _sc_agg_body,
        out_type=(jax.ShapeDtypeStruct((N_NODES, HALF), jnp.float32),
                  jax.ShapeDtypeStruct((N_NODES, HALF), jnp.float32)),
        mesh=plsc.VectorSubcoreMesh(core_axis_name="c", subcore_axis_name="s"),
        scratch_types=[
            pltpu.VMEM((NCHUNK, CHUNK), jnp.int32),
            pltpu.VMEM((NCHUNK, CHUNK), jnp.int32),
            pltpu.VMEM((CHUNK, HALF), jnp.float32),
            pltpu.VMEM_SHARED((N_NODES, HALF), jnp.float32),
            pltpu.SemaphoreType.DMA,
        ],
    )


def _sc_agg(row3, col3, ha, hb):
    return _sc_agg_kernel()(row3, col3, ha, hb)


def _dotT(a, b):
    return lax.dot_general(a, b, (((0,), (0,)), ((), ())),
                           preferred_element_type=jnp.float32,
                           precision=lax.Precision.HIGHEST)


def _dot(a, b):
    return jnp.dot(a, b, preferred_element_type=jnp.float32,
                   precision=lax.Precision.HIGHEST)


def _norm(t, st, i, g, b):
    mean = st[2 * i:2 * i + 1, :] / N_NODES
    var = st[2 * i + 1:2 * i + 2, :] / N_NODES - mean * mean
    return g * (t - mean) * lax.rsqrt(var + 1e-5) + b


def _acc_stats(st, i, r, t):
    s = jnp.sum(t, 0, keepdims=True)
    q = jnp.sum(t * t, 0, keepdims=True)

    @pl.when(r == 0)
    def _():
        st[2 * i:2 * i + 1, :] = s
        st[2 * i + 1:2 * i + 2, :] = q

    @pl.when(r > 0)
    def _():
        st[2 * i:2 * i + 1, :] = st[2 * i:2 * i + 1, :] + s
        st[2 * i + 1:2 * i + 2, :] = st[2 * i + 1:2 * i + 2, :] + q


_blk_h = pl.BlockSpec((RB, HALF), lambda p, r: (jnp.where(p == 0, r, 0), 0))
_blk_o = pl.BlockSpec((RB, HALF), lambda p, r: (jnp.where(p == 2, r, 0), 0))
_cst2 = lambda *shape: pl.BlockSpec(shape, lambda p, r: (0,) * len(shape))


def _tc_mlp_body(h0, h1, a0, a1, eps, W1, b1, g1, be1, W2, b2, g2, be2,
                 o0, o1, t1s, t2s, st):
    p = pl.program_id(0)
    r = pl.program_id(1)
    off = pl.multiple_of(r * RB, 8)

    @pl.when(p == 0)
    def _():
        e = 1.0 + eps[...]
        x0 = e * h0[...] + a0[...]
        x1 = e * h1[...] + a1[...]
        W1v = W1[...]
        t = _dot(x0, W1v[:HALF]) + _dot(x1, W1v[HALF:]) + b1[...]
        t1s[pl.ds(off, RB), :] = t
        _acc_stats(st, 0, r, t)

    @pl.when(p == 1)
    def _():
        t = jnp.maximum(_norm(t1s[pl.ds(off, RB), :], st, 0,
                              g1[...], be1[...]), 0.0)
        u = _dot(t, W2[...]) + b2[...]
        t2s[pl.ds(off, RB), :] = u
        _acc_stats(st, 1, r, u)

    @pl.when(p == 2)
    def _():
        hh = jnp.maximum(_norm(t2s[pl.ds(off, RB), :], st, 1,
                               g2[...], be2[...]), 0.0)
        o0[...] = hh[:, :HALF]
        o1[...] = hh[:, HALF:]


_h2 = (jax.ShapeDtypeStruct((N_NODES, HALF), jnp.float32),
       jax.ShapeDtypeStruct((N_NODES, HALF), jnp.float32))

_tc_mlp = pl.pallas_call(
    _tc_mlp_body,
    grid=(3, NB),
    in_specs=[_blk_h, _blk_h, _blk_h, _blk_h, _cst2(1, 1), _cst2(HID, HID),
              _cst2(1, HID), _cst2(1, HID), _cst2(1, HID), _cst2(HID, HID),
              _cst2(1, HID), _cst2(1, HID), _cst2(1, HID)],
    out_specs=(_blk_o, _blk_o),
    out_shape=_h2,
    scratch_shapes=[pltpu.VMEM((N_NODES, HID), jnp.float32),
                    pltpu.VMEM((N_NODES, HID), jnp.float32),
                    pltpu.VMEM((8, HID), jnp.float32)],
)


_blk_h03 = pl.BlockSpec(
    (RB, HALF), lambda p, r: (jnp.where((p == 0) | (p == 3), r, 0), 0))
_blk_b02 = pl.BlockSpec(
    (RB, 1), lambda p, r: (jnp.where((p == 0) | (p == 2), r, 0), 0))
_blk_o3 = pl.BlockSpec((RB, HALF), lambda p, r: (jnp.where(p == 3, r, 0), 0))


def _tc_vn_body(h0, h1, batch_ref, vne_ref,
                nW, nb, ng, nbe, vW, vb, vg, vbe,
                o0, o1, vne_out, sms, cts, vns, t2s, st):
    p = pl.program_id(0)
    r = pl.program_id(1)
    off = pl.multiple_of(r * RB, 8)

    @pl.when(p == 0)
    def _():
        oh = (lax.broadcasted_iota(jnp.int32, (RB, NGRAPHS), 1)
              == batch_ref[...]).astype(jnp.float32)
        sm0 = _dotT(oh, h0[...])
        sm1 = _dotT(oh, h1[...])
        ct = _dotT(oh, jnp.ones((RB, 8), jnp.float32))

        @pl.when(r == 0)
        def _():
            sms[:, :HALF] = sm0
            sms[:, HALF:] = sm1
            cts[...] = ct

        @pl.when(r > 0)
        def _():
            sms[:, :HALF] = sms[:, :HALF] + sm0
            sms[:, HALF:] = sms[:, HALF:] + sm1
            cts[...] = cts[...] + ct

    @pl.when((p == 1) & (r == 0))
    def _():
        n2v = sms[...] / jnp.maximum(cts[:, 0:1], 1.0)
        t = _dot(n2v, nW[...]) + nb[...]
        m = jnp.mean(t, 0, keepdims=True)
        tc = t - m
        v = jnp.mean(tc * tc, 0, keepdims=True)
        t = jnp.maximum(ng[...] * tc * lax.rsqrt(v + 1e-5) + nbe[...], 0.0)
        vne = vne_ref[...] + t
        vns[...] = vne
        vne_out[...] = vne

    @pl.when(p == 2)
    def _():
        oh = (lax.broadcasted_iota(jnp.int32, (RB, NGRAPHS), 1)
              == batch_ref[...]).astype(jnp.float32)
        v2n = _dot(oh, vns[...])
        t2 = _dot(v2n, vW[...]) + vb[...]
        t2s[pl.ds(off, RB), :] = t2
        _acc_stats(st, 0, r, t2)

    @pl.when(p == 3)
    def _():
        t2 = jnp.maximum(_norm(t2s[pl.ds(off, RB), :], st, 0,
                               vg[...], vbe[...]), 0.0)
        o0[...] = h0[...] + t2[:, :HALF]
        o1[...] = h1[...] + t2[:, HALF:]


_tc_vn = pl.pallas_call(
    _tc_vn_body,
    grid=(4, NB),
    in_specs=[_blk_h03, _blk_h03, _blk_b02, _cst2(NGRAPHS, HID),
              _cst2(HID, HID), _cst2(1, HID), _cst2(1, HID), _cst2(1, HID),
              _cst2(HID, HID), _cst2(1, HID), _cst2(1, HID), _cst2(1, HID)],
    out_specs=(_blk_o3, _blk_o3, _cst2(NGRAPHS, HID)),
    out_shape=_h2 + (jax.ShapeDtypeStruct((NGRAPHS, HID), jnp.float32),),
    scratch_shapes=[pltpu.VMEM((NGRAPHS, HID), jnp.float32),
                    pltpu.VMEM((NGRAPHS, 8), jnp.float32),
                    pltpu.VMEM((NGRAPHS, HID), jnp.float32),
                    pltpu.VMEM((N_NODES, HID), jnp.float32),
                    pltpu.VMEM((8, HID), jnp.float32)],
)


_blk1_h = pl.BlockSpec((RB, HALF), lambda r: (r, 0))
_blk1_b = pl.BlockSpec((RB, 1), lambda r: (r, 0))
_cst1 = lambda *shape: pl.BlockSpec(shape, lambda r: (0,) * len(shape))


def _tc_embed_body(x_ref, emb_ref, pw_ref, pb_ref, h0_ref, h1_ref):
    oh = (lax.broadcasted_iota(jnp.int32, (RB, NTYPES), 1)
          == x_ref[...]).astype(jnp.float32)
    table = _dot(emb_ref[...], pw_ref[...])
    h = _dot(oh, table) + pb_ref[...]
    h0_ref[...] = h[:, :HALF]
    h1_ref[...] = h[:, HALF:]


_tc_embed = pl.pallas_call(
    _tc_embed_body,
    grid=(NB,),
    in_specs=[_blk1_b, _cst1(NTYPES, AF), _cst1(AF, HID), _cst1(1, HID)],
    out_specs=(_blk1_h, _blk1_h),
    out_shape=_h2,
)


def _tc_pred_body(h0, h1, batch_ref, pW1, pb1, pW2, pb2, out_ref, ges):
    r = pl.program_id(0)
    oh = (lax.broadcasted_iota(jnp.int32, (RB, NGRAPHS), 1)
          == batch_ref[...]).astype(jnp.float32)
    ge0 = _dotT(oh, h0[...])
    ge1 = _dotT(oh, h1[...])

    @pl.when(r == 0)
    def _():
        ges[:, :HALF] = ge0
        ges[:, HALF:] = ge1

    @pl.when(r > 0)
    def _():
        ges[:, :HALF] = ges[:, :HALF] + ge0
        ges[:, HALF:] = ges[:, HALF:] + ge1

    @pl.when(r == NB - 1)
    def _():
        t = jnp.maximum(_dot(ges[...], pW1[...]) + pb1[...], 0.0)
        out_ref[...] = _dot(t, pW2[...]) + pb2[...]


_tc_pred = pl.pallas_call(
    _tc_pred_body,
    grid=(NB,),
    in_specs=[_blk1_h, _blk1_h, _blk1_b, _cst1(HID, HID // 2),
              _cst1(1, HID // 2), _cst1(HID // 2, 1), _cst1(1, 1)],
    out_specs=_cst1(NGRAPHS, 1),
    out_shape=jax.ShapeDtypeStruct((NGRAPHS, 1), jnp.float32),
    scratch_shapes=[pltpu.VMEM((NGRAPHS, HID), jnp.float32)],
)


def kernel(x, edge_index, batch, params):
    x2 = x.reshape(N_NODES, 1).astype(jnp.int32)
    batch2 = batch.reshape(N_NODES, 1).astype(jnp.int32)
    row3 = edge_index[0].reshape(NTILES, NCHUNK, CHUNK).astype(jnp.int32)
    col3 = edge_index[1].reshape(NTILES, NCHUNK, CHUNK).astype(jnp.int32)

    h0, h1 = _tc_embed(x2, params["atom_embedding"], params["proj_W"],
                       params["proj_b"].reshape(1, HID))
    vne = jnp.broadcast_to(params["vn_embedding"], (NGRAPHS, HID))

    for i in range(NLAYERS):
        p = params["gin"][i]
        a0, a1 = _sc_agg(row3, col3, h0, h1)
        h0, h1 = _tc_mlp(h0, h1, a0, a1, p["eps"].reshape(1, 1), p["W1"],
                         p["b1"].reshape(1, HID), p["g1"].reshape(1, HID),
                         p["be1"].reshape(1, HID), p["W2"],
                         p["b2"].reshape(1, HID), p["g2"].reshape(1, HID),
                         p["be2"].reshape(1, HID))
        if i < NLAYERS - 1:
            q = params["vn"][i]
            h0, h1, vne = _tc_vn(
                h0, h1, batch2, vne,
                q["nW"], q["nb"].reshape(1, HID), q["ng"].reshape(1, HID),
                q["nbe"].reshape(1, HID),
                q["vW"], q["vb"].reshape(1, HID), q["vg"].reshape(1, HID),
                q["vbe"].reshape(1, HID))
        else:
            out = _tc_pred(
                h0, h1, batch2,
                params["pred_W1"], params["pred_b1"].reshape(1, HID // 2),
                params["pred_W2"], params["pred_b2"].reshape(1, 1))
    return out


# speedup vs baseline: 7.2919x; 1.7456x over previous
import functools

import jax
import jax.numpy as jnp
from jax import lax
from jax.experimental import pallas as pl
from jax.experimental.pallas import tpu as pltpu
from jax.experimental.pallas import tpu_sc as plsc

NTYPES = 100
AF = 128
HID = 256
HALF = 128
NLAYERS = 5
N_NODES = 10000
N_EDGES = 160000
NGRAPHS = 64

NTILES = 16
EPT = N_EDGES // NTILES
CHUNK = 80
NCHUNK = EPT // CHUNK
ROWS_A = 624
ROWS_LAST = N_NODES - (NTILES - 1) * ROWS_A

RB = 1000
NB = N_NODES // RB


def _sc_agg_body(row_hbm, col_hbm, ha_hbm, hb_hbm, outa_hbm, outb_hbm,
                 colv, rowb, rowb2, gbuf, gbuf2, agg_sh, sem, sem2,
                 rsem, rsem2):
    c = lax.axis_index("c")
    s = lax.axis_index("s")

    def zrow(r, carry):
        for k in range(HALF // 16):
            gbuf[r, pl.ds(k * 16, 16)] = jnp.zeros((16,), jnp.float32)
        return carry
    lax.fori_loop(0, CHUNK, zrow, 0)
    base = s * ROWS_A
    for b in range(ROWS_A // CHUNK):
        pltpu.sync_copy(gbuf, agg_sh.at[pl.ds(base + b * CHUNK, CHUNK)])
    pltpu.sync_copy(gbuf.at[pl.ds(0, ROWS_A - 560)],
                    agg_sh.at[pl.ds(base + 560, ROWS_A - 560)])

    @pl.when(s == NTILES - 1)
    def _():
        pltpu.sync_copy(gbuf.at[pl.ds(0, ROWS_LAST - ROWS_A)],
                        agg_sh.at[pl.ds(base + ROWS_A, ROWS_LAST - ROWS_A)])
    plsc.subcore_barrier()

    pltpu.sync_copy(col_hbm.at[s], colv)
    ebase = s * EPT

    def fire(j, buf, bsem, rb, rs):
        @pl.when(c == 0)
        def _():
            pltpu.async_copy(ha_hbm.at[colv.at[j]], buf, bsem)

        @pl.when(c == 1)
        def _():
            pltpu.async_copy(hb_hbm.at[colv.at[j]], buf, bsem)
        pltpu.async_copy(row_hbm.at[pl.ds(ebase + j * CHUNK, CHUNK)], rb, rs)

    def drain_scatter(j, buf, bsem, rb, rs):
        pltpu.make_async_copy(ha_hbm.at[colv.at[j]], buf, bsem).wait()
        pltpu.make_async_copy(row_hbm.at[pl.ds(ebase + j * CHUNK, CHUNK)],
                              rb, rs).wait()
        pltpu.sync_copy(buf, agg_sh.at[rb], add=True)

    fire(0, gbuf, sem, rowb, rsem)

    def chunk(j, carry):
        @pl.when(j % 2 == 0)
        def _():
            @pl.when(j + 1 < NCHUNK)
            def _():
                fire(j + 1, gbuf2, sem2, rowb2, rsem2)
            drain_scatter(j, gbuf, sem, rowb, rsem)

        @pl.when(j % 2 == 1)
        def _():
            @pl.when(j + 1 < NCHUNK)
            def _():
                fire(j + 1, gbuf, sem, rowb, rsem)
            drain_scatter(j, gbuf2, sem2, rowb2, rsem2)
        return carry
    lax.fori_loop(0, NCHUNK, chunk, 0)
    plsc.subcore_barrier()

    @pl.when((c == 0) & (s != NTILES - 1))
    def _():
        pltpu.sync_copy(agg_sh.at[pl.ds(base, ROWS_A)],
                        outa_hbm.at[pl.ds(base, ROWS_A)])

    @pl.when((c == 0) & (s == NTILES - 1))
    def _():
        pltpu.sync_copy(agg_sh.at[pl.ds(base, ROWS_LAST)],
                        outa_hbm.at[pl.ds(base, ROWS_LAST)])

    @pl.when((c == 1) & (s != NTILES - 1))
    def _():
        pltpu.sync_copy(agg_sh.at[pl.ds(base, ROWS_A)],
                        outb_hbm.at[pl.ds(base, ROWS_A)])

    @pl.when((c == 1) & (s == NTILES - 1))
    def _():
        pltpu.sync_copy(agg_sh.at[pl.ds(base, ROWS_LAST)],
                        outb_hbm.at[pl.ds(base, ROWS_LAST)])


@functools.cache
def _sc_agg_kernel():
    return pl.kernel(
        _sc_agg_body,
        out_type=(jax.ShapeDtypeStruct((N_NODES, HALF), jnp.float32),
                  jax.ShapeDtypeStruct((N_NODES, HALF), jnp.float32)),
        mesh=plsc.VectorSubcoreMesh(core_axis_name="c", subcore_axis_name="s"),
        scratch_types=[
            pltpu.VMEM((NCHUNK, CHUNK), jnp.int32),
            pltpu.VMEM((CHUNK,), jnp.int32),
            pltpu.VMEM((CHUNK,), jnp.int32),
            pltpu.VMEM((CHUNK, HALF), jnp.float32),
            pltpu.VMEM((CHUNK, HALF), jnp.float32),
            pltpu.VMEM_SHARED((N_NODES, HALF), jnp.float32),
            pltpu.SemaphoreType.DMA,
            pltpu.SemaphoreType.DMA,
            pltpu.SemaphoreType.DMA,
            pltpu.SemaphoreType.DMA,
        ],
    )


def _sc_agg(row3, col3, ha, hb):
    return _sc_agg_kernel()(row3, col3, ha, hb)


def _dotT(a, b):
    return lax.dot_general(a, b, (((0,), (0,)), ((), ())),
                           preferred_element_type=jnp.float32,
                           precision=lax.Precision.HIGHEST)


def _dsel(a, b):
    return jnp.dot(a, b, preferred_element_type=jnp.float32,
                   precision=lax.Precision.HIGHEST)


def _dot(a, b):
    return jnp.dot(a, b, preferred_element_type=jnp.float32)


def _norm(t, st, i, g, b):
    mean = st[2 * i:2 * i + 1, :] / N_NODES
    var = st[2 * i + 1:2 * i + 2, :] / N_NODES - mean * mean
    return g * (t - mean) * lax.rsqrt(var + 1e-5) + b


def _acc_stats(st, i, r, t):
    s = jnp.sum(t, 0, keepdims=True)
    q = jnp.sum(t * t, 0, keepdims=True)

    @pl.when(r == 0)
    def _():
        st[2 * i:2 * i + 1, :] = s
        st[2 * i + 1:2 * i + 2, :] = q

    @pl.when(r > 0)
    def _():
        st[2 * i:2 * i + 1, :] = st[2 * i:2 * i + 1, :] + s
        st[2 * i + 1:2 * i + 2, :] = st[2 * i + 1:2 * i + 2, :] + q


_blk_in = pl.BlockSpec((RB, HALF), lambda p, r: (jnp.where(p == 0, r, 0), 0))
_blk_b24 = pl.BlockSpec(
    (RB, 1), lambda p, r: (jnp.where((p == 2) | (p == 4), r, 0), 0))
_blk_o5 = pl.BlockSpec((RB, HALF), lambda p, r: (jnp.where(p == 5, r, 0), 0))
_cst2 = lambda *shape: pl.BlockSpec(shape, lambda p, r: (0,) * len(shape))

_h2 = (jax.ShapeDtypeStruct((N_NODES, HALF), jnp.float32),
       jax.ShapeDtypeStruct((N_NODES, HALF), jnp.float32))


def _mlp_phases(p, r, off, h0, h1, a0, a1, eps, W1, b1, g1, be1, W2, b2,
                g2, be2, t1s, t2s, st):
    @pl.when(p == 0)
    def _():
        e = 1.0 + eps[...]
        x = jnp.concatenate([e * h0[...] + a0[...],
                             e * h1[...] + a1[...]], axis=1)
        t = _dot(x, W1[...]) + b1[...]
        t1s[pl.ds(off, RB), :] = t
        _acc_stats(st, 0, r, t)

    @pl.when(p == 1)
    def _():
        t = jnp.maximum(_norm(t1s[pl.ds(off, RB), :], st, 0,
                              g1[...], be1[...]), 0.0)
        u = _dot(t, W2[...]) + b2[...]
        t2s[pl.ds(off, RB), :] = u
        _acc_stats(st, 1, r, u)


def _onehot(batch_ref):
    return (lax.broadcasted_iota(jnp.int32, (RB, NGRAPHS), 1)
            == batch_ref[...]).astype(jnp.float32)


def _tc_layer_body(h0, h1, a0, a1, batch_ref, vne_ref,
                   eps, W1, b1, g1, be1, W2, b2, g2, be2,
                   nW, nb, ng, nbe, vW, vb, vg, vbe,
                   o0, o1, vne_out, t1s, t2s, st, sms, cts, vns):
    p = pl.program_id(0)
    r = pl.program_id(1)
    off = pl.multiple_of(r * RB, 8)
    _mlp_phases(p, r, off, h0, h1, a0, a1, eps, W1, b1, g1, be1, W2, b2,
                g2, be2, t1s, t2s, st)

    @pl.when(p == 2)
    def _():
        hh = jnp.maximum(_norm(t2s[pl.ds(off, RB), :], st, 1,
                               g2[...], be2[...]), 0.0)
        t1s[pl.ds(off, RB), :] = hh
        oh = _onehot(batch_ref)
        sm = _dotT(oh, hh)
        ct = _dotT(oh, jnp.ones((RB, 8), jnp.float32))

        @pl.when(r == 0)
        def _():
            sms[...] = sm
            cts[...] = ct

        @pl.when(r > 0)
        def _():
            sms[...] = sms[...] + sm
            cts[...] = cts[...] + ct

    @pl.when((p == 3) & (r == 0))
    def _():
        n2v = sms[...] / jnp.maximum(cts[:, 0:1], 1.0)
        t = _dot(n2v, nW[...]) + nb[...]
        m = jnp.mean(t, 0, keepdims=True)
        tc = t - m
        v = jnp.mean(tc * tc, 0, keepdims=True)
        t = jnp.maximum(ng[...] * tc * lax.rsqrt(v + 1e-5) + nbe[...], 0.0)
        vne = vne_ref[...] + t
        vns[...] = vne
        vne_out[...] = vne

    @pl.when(p == 4)
    def _():
        oh = _onehot(batch_ref)
        v2n = _dsel(oh, vns[...])
        t2 = _dot(v2n, vW[...]) + vb[...]
        t2s[pl.ds(off, RB), :] = t2
        _acc_stats(st, 2, r, t2)

    @pl.when(p == 5)
    def _():
        t2 = jnp.maximum(_norm(t2s[pl.ds(off, RB), :], st, 2,
                               vg[...], vbe[...]), 0.0)
        hn = t1s[pl.ds(off, RB), :] + t2
        o0[...] = hn[:, :HALF]
        o1[...] = hn[:, HALF:]


_tc_layer = pl.pallas_call(
    _tc_layer_body,
    grid=(6, NB),
    in_specs=[_blk_in, _blk_in, _blk_in, _blk_in, _blk_b24,
              _cst2(NGRAPHS, HID),
              _cst2(1, 1), _cst2(HID, HID), _cst2(1, HID), _cst2(1, HID),
              _cst2(1, HID), _cst2(HID, HID), _cst2(1, HID), _cst2(1, HID),
              _cst2(1, HID),
              _cst2(HID, HID), _cst2(1, HID), _cst2(1, HID), _cst2(1, HID),
              _cst2(HID, HID), _cst2(1, HID), _cst2(1, HID), _cst2(1, HID)],
    out_specs=(_blk_o5, _blk_o5, _cst2(NGRAPHS, HID)),
    out_shape=_h2 + (jax.ShapeDtypeStruct((NGRAPHS, HID), jnp.float32),),
    scratch_shapes=[pltpu.VMEM((N_NODES, HID), jnp.float32),
                    pltpu.VMEM((N_NODES, HID), jnp.float32),
                    pltpu.VMEM((8, HID), jnp.float32),
                    pltpu.VMEM((NGRAPHS, HID), jnp.float32),
                    pltpu.VMEM((NGRAPHS, 8), jnp.float32),
                    pltpu.VMEM((NGRAPHS, HID), jnp.float32)],
)


_blk_b2 = pl.BlockSpec((RB, 1), lambda p, r: (jnp.where(p == 2, r, 0), 0))


def _tc_final_body(h0, h1, a0, a1, batch_ref,
                   eps, W1, b1, g1, be1, W2, b2, g2, be2,
                   pW1, pb1, pW2, pb2, out_ref, t1s, t2s, st, ges):
    p = pl.program_id(0)
    r = pl.program_id(1)
    off = pl.multiple_of(r * RB, 8)
    _mlp_phases(p, r, off, h0, h1, a0, a1, eps, W1, b1, g1, be1, W2, b2,
                g2, be2, t1s, t2s, st)

    @pl.when(p == 2)
    def _():
        hh = jnp.maximum(_norm(t2s[pl.ds(off, RB), :], st, 1,
                               g2[...], be2[...]), 0.0)
        oh = _onehot(batch_ref)
        ge = _dotT(oh, hh)

        @pl.when(r == 0)
        def _():
            ges[...] = ge

        @pl.when(r > 0)
        def _():
            ges[...] = ges[...] + ge

        @pl.when(r == NB - 1)
        def _():
            t = jnp.maximum(_dot(ges[...], pW1[...]) + pb1[...], 0.0)
            out_ref[...] = _dot(t, pW2[...]) + pb2[...]


_tc_final = pl.pallas_call(
    _tc_final_body,
    grid=(3, NB),
    in_specs=[_blk_in, _blk_in, _blk_in, _blk_in, _blk_b2,
              _cst2(1, 1), _cst2(HID, HID), _cst2(1, HID), _cst2(1, HID),
              _cst2(1, HID), _cst2(HID, HID), _cst2(1, HID), _cst2(1, HID),
              _cst2(1, HID),
              _cst2(HID, HID // 2), _cst2(1, HID // 2), _cst2(HID // 2, 1),
              _cst2(1, 1)],
    out_specs=_cst2(NGRAPHS, 1),
    out_shape=jax.ShapeDtypeStruct((NGRAPHS, 1), jnp.float32),
    scratch_shapes=[pltpu.VMEM((N_NODES, HID), jnp.float32),
                    pltpu.VMEM((N_NODES, HID), jnp.float32),
                    pltpu.VMEM((8, HID), jnp.float32),
                    pltpu.VMEM((NGRAPHS, HID), jnp.float32)],
)


_blk1_h = pl.BlockSpec((RB, HALF), lambda r: (r, 0))
_blk1_b = pl.BlockSpec((RB, 1), lambda r: (r, 0))
_cst1 = lambda *shape: pl.BlockSpec(shape, lambda r: (0,) * len(shape))


def _tc_embed_body(x_ref, emb_ref, pw_ref, pb_ref, h0_ref, h1_ref):
    oh = (lax.broadcasted_iota(jnp.int32, (RB, NTYPES), 1)
          == x_ref[...]).astype(jnp.float32)
    he = _dsel(oh, emb_ref[...])
    h = _dot(he, pw_ref[...]) + pb_ref[...]
    h0_ref[...] = h[:, :HALF]
    h1_ref[...] = h[:, HALF:]


_tc_embed = pl.pallas_call(
    _tc_embed_body,
    grid=(NB,),
    in_specs=[_blk1_b, _cst1(NTYPES, AF), _cst1(AF, HID), _cst1(1, HID)],
    out_specs=(_blk1_h, _blk1_h),
    out_shape=_h2,
)


def kernel(x, edge_index, batch, params):
    x2 = x.reshape(N_NODES, 1).astype(jnp.int32)
    batch2 = batch.reshape(N_NODES, 1).astype(jnp.int32)
    row3 = edge_index[0].astype(jnp.int32)
    col3 = edge_index[1].reshape(NTILES, NCHUNK, CHUNK).astype(jnp.int32)

    h0, h1 = _tc_embed(x2, params["atom_embedding"], params["proj_W"],
                       params["proj_b"].reshape(1, HID))
    vne = jnp.broadcast_to(params["vn_embedding"], (NGRAPHS, HID))

    for i in range(NLAYERS):
        p = params["gin"][i]
        a0, a1 = _sc_agg(row3, col3, h0, h1)
        gin_args = (p["eps"].reshape(1, 1), p["W1"], p["b1"].reshape(1, HID),
                    p["g1"].reshape(1, HID), p["be1"].reshape(1, HID),
                    p["W2"], p["b2"].reshape(1, HID),
                    p["g2"].reshape(1, HID), p["be2"].reshape(1, HID))
        if i < NLAYERS - 1:
            q = params["vn"][i]
            h0, h1, vne = _tc_layer(
                h0, h1, a0, a1, batch2, vne, *gin_args,
                q["nW"], q["nb"].reshape(1, HID), q["ng"].reshape(1, HID),
                q["nbe"].reshape(1, HID),
                q["vW"], q["vb"].reshape(1, HID), q["vg"].reshape(1, HID),
                q["vbe"].reshape(1, HID))
        else:
            out = _tc_final(
                h0, h1, a0, a1, batch2, *gin_args,
                params["pred_W1"], params["pred_b1"].reshape(1, HID // 2),
                params["pred_W2"], params["pred_b2"].reshape(1, 1))
    return out
